# Initial kernel scaffold; baseline (speedup 1.0000x reference)
import jax
import jax.numpy as jnp
from jax.experimental import pallas as pl


def kernel(x, batch_size, pos_table, ln_w, ln_b):
    raise NotImplementedError("write your pallas kernel here")


# speedup vs baseline: 2.6475x; 2.6475x over previous
import functools

import jax
import jax.numpy as jnp
from jax.experimental import pallas as pl
from jax.experimental.pallas import tpu as pltpu

BS = 256


def _ln_kernel(x_ref, p_ref, w_ref, b_ref, o_ref):
    e = x_ref[0] + p_ref[...]
    mean = jnp.mean(e, axis=1, keepdims=True)
    c = e - mean
    var = jnp.mean(c * c, axis=1, keepdims=True)
    inv = jax.lax.rsqrt(var + 1e-5)
    o_ref[0] = (c * inv) * w_ref[0] + b_ref[0]


@functools.partial(jax.jit, static_argnames=("batch_size",))
def _run(x, pos_table, ln_w, ln_b, batch_size):
    B, S, D = x.shape
    grid = (B, S // BS)
    return pl.pallas_call(
        _ln_kernel,
        grid=grid,
        in_specs=[
            pl.BlockSpec((1, BS, D), lambda b, s: (b, s, 0)),
            pl.BlockSpec((BS, D), lambda b, s: (s, 0)),
            pl.BlockSpec((1, D), lambda b, s: (0, 0)),
            pl.BlockSpec((1, D), lambda b, s: (0, 0)),
        ],
        out_specs=pl.BlockSpec((1, BS, D), lambda b, s: (b, s, 0)),
        out_shape=jax.ShapeDtypeStruct((B, S, D), x.dtype),
        compiler_params=pltpu.CompilerParams(
            dimension_semantics=("parallel", "arbitrary"),
        ),
    )(x, pos_table, ln_w.reshape(1, D), ln_b.reshape(1, D))


def kernel(x, batch_size, pos_table, ln_w, ln_b):
    return _run(x, pos_table, ln_w, ln_b, int(x.shape[0]))


# speedup vs baseline: 2.7095x; 1.0234x over previous
import functools

import jax
import jax.numpy as jnp
from jax.experimental import pallas as pl
from jax.experimental.pallas import tpu as pltpu

BS = 256


def _ln_kernel(x_ref, p_ref, w_ref, b_ref, o_ref):
    e = x_ref[0] + p_ref[...]
    mean = jnp.mean(e, axis=1, keepdims=True)
    c = e - mean
    var = jnp.mean(c * c, axis=1, keepdims=True)
    inv = jax.lax.rsqrt(var + 1e-5)
    o_ref[0] = (c * inv) * w_ref[0] + b_ref[0]


@functools.partial(jax.jit, static_argnames=("batch_size",))
def _run(x, pos_table, ln_w, ln_b, batch_size):
    B, S, D = x.shape
    grid = (S // BS, B)
    return pl.pallas_call(
        _ln_kernel,
        grid=grid,
        in_specs=[
            pl.BlockSpec((1, BS, D), lambda s, b: (b, s, 0)),
            pl.BlockSpec((BS, D), lambda s, b: (s, 0)),
            pl.BlockSpec((1, D), lambda s, b: (0, 0)),
            pl.BlockSpec((1, D), lambda s, b: (0, 0)),
        ],
        out_specs=pl.BlockSpec((1, BS, D), lambda s, b: (b, s, 0)),
        out_shape=jax.ShapeDtypeStruct((B, S, D), x.dtype),
        compiler_params=pltpu.CompilerParams(
            dimension_semantics=("arbitrary", "arbitrary"),
        ),
    )(x, pos_table, ln_w.reshape(1, D), ln_b.reshape(1, D))


def kernel(x, batch_size, pos_table, ln_w, ln_b):
    return _run(x, pos_table, ln_w, ln_b, int(x.shape[0]))


# speedup vs baseline: 4.3858x; 1.6187x over previous
import functools

import jax
import jax.numpy as jnp
from jax.experimental import pallas as pl
from jax.experimental.pallas import tpu as pltpu

BS = 256


def _ln_kernel(x_ref, p_ref, w_ref, b_ref, o_ref):
    e = x_ref[...] + p_ref[None]
    mean = jnp.mean(e, axis=-1, keepdims=True)
    c = e - mean
    var = jnp.mean(c * c, axis=-1, keepdims=True)
    inv = jax.lax.rsqrt(var + 1e-5)
    o_ref[...] = (c * inv) * w_ref[0] + b_ref[0]


@functools.partial(jax.jit, static_argnames=("batch_size",))
def _run(x, pos_table, ln_w, ln_b, batch_size):
    B, S, D = x.shape
    grid = (S // BS,)
    return pl.pallas_call(
        _ln_kernel,
        grid=grid,
        in_specs=[
            pl.BlockSpec((B, BS, D), lambda s: (0, s, 0)),
            pl.BlockSpec((BS, D), lambda s: (s, 0)),
            pl.BlockSpec((1, D), lambda s: (0, 0)),
            pl.BlockSpec((1, D), lambda s: (0, 0)),
        ],
        out_specs=pl.BlockSpec((B, BS, D), lambda s: (0, s, 0)),
        out_shape=jax.ShapeDtypeStruct((B, S, D), x.dtype),
        compiler_params=pltpu.CompilerParams(
            dimension_semantics=("arbitrary",),
        ),
    )(x, pos_table, ln_w.reshape(1, D), ln_b.reshape(1, D))


def kernel(x, batch_size, pos_table, ln_w, ln_b):
    return _run(x, pos_table, ln_w, ln_b, int(x.shape[0]))


# speedup vs baseline: 4.4822x; 1.0220x over previous
import functools

import jax
import jax.numpy as jnp
from jax.experimental import pallas as pl
from jax.experimental.pallas import tpu as pltpu

BS = 512


def _ln_kernel(x_ref, p_ref, w_ref, b_ref, o_ref):
    e = x_ref[...] + p_ref[None]
    mean = jnp.mean(e, axis=-1, keepdims=True)
    c = e - mean
    var = jnp.mean(c * c, axis=-1, keepdims=True)
    inv = jax.lax.rsqrt(var + 1e-5)
    o_ref[...] = (c * inv) * w_ref[0] + b_ref[0]


@functools.partial(jax.jit, static_argnames=("batch_size",))
def _run(x, pos_table, ln_w, ln_b, batch_size):
    B, S, D = x.shape
    grid = (S // BS,)
    return pl.pallas_call(
        _ln_kernel,
        grid=grid,
        in_specs=[
            pl.BlockSpec((B, BS, D), lambda s: (0, s, 0)),
            pl.BlockSpec((BS, D), lambda s: (s, 0)),
            pl.BlockSpec((1, D), lambda s: (0, 0)),
            pl.BlockSpec((1, D), lambda s: (0, 0)),
        ],
        out_specs=pl.BlockSpec((B, BS, D), lambda s: (0, s, 0)),
        out_shape=jax.ShapeDtypeStruct((B, S, D), x.dtype),
        compiler_params=pltpu.CompilerParams(
            dimension_semantics=("arbitrary",),
        ),
    )(x, pos_table, ln_w.reshape(1, D), ln_b.reshape(1, D))


def kernel(x, batch_size, pos_table, ln_w, ln_b):
    return _run(x, pos_table, ln_w, ln_b, int(x.shape[0]))
